# baseline (speedup 1.0000x reference)
import jax
import jax.numpy as jnp
from jax.experimental import pallas as pl
from jax.experimental.pallas import tpu as pltpu

_L = 4096
_DM = 256
_DH = 128
_DS = 16
_DTR = 16
_CHUNK = 512
_NC = _L // _CHUNK
_SCHUNK = 128
_NSC = _L // _SCHUNK

_HP = jax.lax.Precision.HIGHEST
_DP = jax.lax.Precision.DEFAULT


def _silu(v):
    return v * jax.lax.logistic(v)


def _mamba_body(u_ref, w_in_ref, w_dl_ref, w_bc_ref, w_out_ref, A_ref, aux_ref,
                o_ref, xz_s, xf_s, zf_s, dl_s, dxu_s, bct_s, y_s, a_s):
    for k in range(_NC):
        sl = slice(k * _CHUNK, (k + 1) * _CHUNK)
        xz_s[sl, :] = jnp.dot(u_ref[0, sl, :], w_in_ref[...], precision=_DP)

    w0x = aux_ref[0:1, :]
    w1x = aux_ref[1:2, :]
    w2x = aux_ref[2:3, :]
    w0z = aux_ref[3:4, :]
    w1z = aux_ref[4:5, :]
    w2z = aux_ref[5:6, :]
    zrow = jnp.zeros((1, _DH), jnp.float32)
    for k in range(_NC):
        sl = slice(k * _CHUNK, (k + 1) * _CHUNK)
        xc = xz_s[sl, 0:_DH]
        zc = xz_s[sl, _DH:_DM]
        if k == 0:
            px, pz = zrow, zrow
        else:
            prev = xz_s[k * _CHUNK - 8:k * _CHUNK, :]
            px, pz = prev[7:8, 0:_DH], prev[7:8, _DH:_DM]
        if k == _NC - 1:
            nx, nz = zrow, zrow
        else:
            nxt = xz_s[(k + 1) * _CHUNK:(k + 1) * _CHUNK + 8, :]
            nx, nz = nxt[0:1, 0:_DH], nxt[0:1, _DH:_DM]
        xl = jnp.concatenate([px, xc[:-1]], axis=0)
        xr = jnp.concatenate([xc[1:], nx], axis=0)
        zl = jnp.concatenate([pz, zc[:-1]], axis=0)
        zr = jnp.concatenate([zc[1:], nz], axis=0)
        xf_s[sl, :] = _silu(w0x * xl + w1x * xc + w2x * xr)
        zf_s[sl, :] = _silu(w0z * zl + w1z * zc + w2z * zr)

    dtb = aux_ref[6:7, :]
    for k in range(_NC):
        sl = slice(k * _CHUNK, (k + 1) * _CHUNK)
        xfc = xf_s[sl, :]
        dp = jnp.dot(xfc, w_dl_ref[...], precision=_HP) + dtb
        dl = jnp.where(dp > 20.0, dp, jnp.log1p(jnp.exp(jnp.minimum(dp, 20.0))))
        dl_s[sl, :] = dl
        dxu_s[sl, :] = dl * xfc

    bct_s[...] = jax.lax.dot_general(
        w_bc_ref[...], xf_s[...], (((1,), (1,)), ((), ())), precision=_DP)

    A_v = A_ref[...]
    _i0 = jax.lax.broadcasted_iota(jnp.int32, (8, _DS * 8), 0)
    _i1 = jax.lax.broadcasted_iota(jnp.int32, (8, _DS * 8), 1)
    S2_v = jnp.where(_i1 // _DS == _i0, 1.0, 0.0).astype(jnp.float32)

    def _fill_a(slot, cc):
        dblk = dl_s[pl.ds(pl.multiple_of(cc * _SCHUNK, _SCHUNK), _SCHUNK), :]
        for j in range(_SCHUNK):
            a_s[slot, j * _DS:(j + 1) * _DS, :] = jnp.exp(A_v * dblk[j:j + 1, :])

    _fill_a(0, 0)

    def chunk(c, h):
        base = pl.multiple_of(c * _SCHUNK, _SCHUNK)
        par = jax.lax.rem(c, 2)
        xblk = dxu_s[pl.ds(base, _SCHUNK), :]
        Bblk = bct_s[0:_DS, pl.ds(base, _SCHUNK)]
        Cblk = bct_s[_DS:2 * _DS, pl.ds(base, _SCHUNK)]
        _fill_a(1 - par, jnp.minimum(c + 1, _NSC - 1))
        for g in range(_SCHUNK // 8):
            wr = []
            for j in range(g * 8, (g + 1) * 8):
                bj = xblk[j:j + 1, :] * Bblk[:, j:j + 1]
                h = a_s[par, j * _DS:(j + 1) * _DS, :] * h + bj
                wr.append(h * Cblk[:, j:j + 1])
            W_v = jnp.concatenate(wr, axis=0)
            y_s[pl.ds(base + g * 8, 8), :] = jnp.dot(S2_v, W_v, precision=_DP)
        return h

    jax.lax.fori_loop(0, _NSC, chunk, jnp.zeros((_DS, _DH), jnp.float32))

    Dv = aux_ref[7:8, :]
    for k in range(_NC):
        sl = slice(k * _CHUNK, (k + 1) * _CHUNK)
        cat = jnp.concatenate([y_s[sl, :] + xf_s[sl, :] * Dv, zf_s[sl, :]], axis=1)
        o_ref[0, sl, :] = jnp.dot(cat, w_out_ref[...], precision=_DP)


def kernel(u_0, u_1, in_proj_w, conv_x_w, conv_z_w, x_proj_w, dt_proj_w,
           dt_proj_b, A_log, D, out_proj_w):
    b0 = u_0.shape[0]
    u_all = jnp.concatenate([u_0, u_1], axis=0)
    nb = u_all.shape[0]

    w_in_T = in_proj_w.T
    w_dl_T = jnp.dot(x_proj_w[:_DTR].T, dt_proj_w.T, precision=_HP)
    w_bc = x_proj_w[_DTR:]
    w_out_T = out_proj_w.T
    A_T = -jnp.exp(A_log).T
    aux = jnp.stack([conv_x_w[:, 0, 0], conv_x_w[:, 0, 1], conv_x_w[:, 0, 2],
                     conv_z_w[:, 0, 0], conv_z_w[:, 0, 1], conv_z_w[:, 0, 2],
                     dt_proj_b, D], axis=0)

    out_all = pl.pallas_call(
        _mamba_body,
        grid=(nb,),
        in_specs=[
            pl.BlockSpec((1, _L, _DM), lambda i: (i, 0, 0)),
            pl.BlockSpec((_DM, _DM), lambda i: (0, 0)),
            pl.BlockSpec((_DH, _DH), lambda i: (0, 0)),
            pl.BlockSpec((2 * _DS, _DH), lambda i: (0, 0)),
            pl.BlockSpec((_DM, _DM), lambda i: (0, 0)),
            pl.BlockSpec((_DS, _DH), lambda i: (0, 0)),
            pl.BlockSpec((8, _DH), lambda i: (0, 0)),
        ],
        out_specs=pl.BlockSpec((1, _L, _DM), lambda i: (i, 0, 0)),
        out_shape=jax.ShapeDtypeStruct((nb, _L, _DM), jnp.float32),
        scratch_shapes=[
            pltpu.VMEM((_L, _DM), jnp.float32),
            pltpu.VMEM((_L, _DH), jnp.float32),
            pltpu.VMEM((_L, _DH), jnp.float32),
            pltpu.VMEM((_L, _DH), jnp.float32),
            pltpu.VMEM((_L, _DH), jnp.float32),
            pltpu.VMEM((2 * _DS, _L), jnp.float32),
            pltpu.VMEM((_L, _DH), jnp.float32),
            pltpu.VMEM((2, _SCHUNK * _DS, _DH), jnp.float32),
        ],
        compiler_params=pltpu.CompilerParams(
            dimension_semantics=("parallel",),
            vmem_limit_bytes=56 * 1024 * 1024,
        ),
        name="mamba_vision_fused",
    )(u_all, w_in_T, w_dl_T, w_bc, w_out_T, A_T, aux)

    return out_all[:b0], out_all[b0:]


# speedup vs baseline: 1.1121x; 1.1121x over previous
import jax
import jax.numpy as jnp
from jax.experimental import pallas as pl
from jax.experimental.pallas import tpu as pltpu

_L = 4096
_DM = 256
_DH = 128
_DS = 16
_DTR = 16
_CHUNK = 512
_NC = _L // _CHUNK
_SCHUNK = 128
_NSC = _L // _SCHUNK

_HP = jax.lax.Precision.HIGHEST
_DP = jax.lax.Precision.DEFAULT


def _silu(v):
    return v * jax.lax.logistic(v)


def _mamba_body(u_ref, w_in_ref, w_dl_ref, w_bc_ref, w_out_ref, A_ref, aux_ref,
                o_ref, xz_s, xf_s, zf_s, dl_s, dxu_s, bct_s, y_s, a_s):
    for k in range(_NC):
        sl = slice(k * _CHUNK, (k + 1) * _CHUNK)
        xz_s[sl, :] = jnp.dot(u_ref[0, sl, :], w_in_ref[...], precision=_DP)

    w0x = aux_ref[0:1, :]
    w1x = aux_ref[1:2, :]
    w2x = aux_ref[2:3, :]
    w0z = aux_ref[3:4, :]
    w1z = aux_ref[4:5, :]
    w2z = aux_ref[5:6, :]
    zrow = jnp.zeros((1, _DH), jnp.float32)
    for k in range(_NC):
        sl = slice(k * _CHUNK, (k + 1) * _CHUNK)
        xc = xz_s[sl, 0:_DH]
        zc = xz_s[sl, _DH:_DM]
        if k == 0:
            px, pz = zrow, zrow
        else:
            prev = xz_s[k * _CHUNK - 8:k * _CHUNK, :]
            px, pz = prev[7:8, 0:_DH], prev[7:8, _DH:_DM]
        if k == _NC - 1:
            nx, nz = zrow, zrow
        else:
            nxt = xz_s[(k + 1) * _CHUNK:(k + 1) * _CHUNK + 8, :]
            nx, nz = nxt[0:1, 0:_DH], nxt[0:1, _DH:_DM]
        xl = jnp.concatenate([px, xc[:-1]], axis=0)
        xr = jnp.concatenate([xc[1:], nx], axis=0)
        zl = jnp.concatenate([pz, zc[:-1]], axis=0)
        zr = jnp.concatenate([zc[1:], nz], axis=0)
        xf_s[sl, :] = _silu(w0x * xl + w1x * xc + w2x * xr)
        zf_s[sl, :] = _silu(w0z * zl + w1z * zc + w2z * zr)

    dtb = aux_ref[6:7, :]
    for k in range(_NC):
        sl = slice(k * _CHUNK, (k + 1) * _CHUNK)
        xfc = xf_s[sl, :]
        dp = jnp.dot(xfc, w_dl_ref[...], precision=_HP) + dtb
        dl = jnp.where(dp > 20.0, dp,
                       jnp.log(1.0 + jnp.exp(jnp.minimum(dp, 20.0))))
        dl_s[sl, :] = dl
        dxu_s[sl, :] = dl * xfc
        bct_s[:, sl] = jax.lax.dot_general(
            w_bc_ref[...], xfc, (((1,), (1,)), ((), ())), precision=_DP)

    A_v = A_ref[...]

    def _fill_a(slot, cc):
        dblk = dl_s[pl.ds(pl.multiple_of(cc * _SCHUNK, _SCHUNK), _SCHUNK), :]
        for j in range(_SCHUNK):
            a_s[slot, j * _DS:(j + 1) * _DS, :] = jnp.exp(A_v * dblk[j:j + 1, :])

    _fill_a(0, 0)

    def chunk(c, h):
        base = pl.multiple_of(c * _SCHUNK, _SCHUNK)
        par = jax.lax.rem(c, 2)
        xblk = dxu_s[pl.ds(base, _SCHUNK), :]
        Bblk = bct_s[0:_DS, pl.ds(base, _SCHUNK)]
        Cblk = bct_s[_DS:2 * _DS, pl.ds(base, _SCHUNK)]
        _fill_a(1 - par, jnp.minimum(c + 1, _NSC - 1))
        for g in range(_SCHUNK // 8):
            rows = []
            for j in range(g * 8, (g + 1) * 8):
                bj = xblk[j:j + 1, :] * Bblk[:, j:j + 1]
                h = a_s[par, j * _DS:(j + 1) * _DS, :] * h + bj
                rows.append(jnp.sum(h * Cblk[:, j:j + 1], axis=0, keepdims=True))
            y_s[pl.ds(base + g * 8, 8), :] = jnp.concatenate(rows, axis=0)
        return h

    jax.lax.fori_loop(0, _NSC, chunk, jnp.zeros((_DS, _DH), jnp.float32))

    Dv = aux_ref[7:8, :]
    for k in range(_NC):
        sl = slice(k * _CHUNK, (k + 1) * _CHUNK)
        cat = jnp.concatenate([y_s[sl, :] + xf_s[sl, :] * Dv, zf_s[sl, :]], axis=1)
        o_ref[0, sl, :] = jnp.dot(cat, w_out_ref[...], precision=_DP)


def kernel(u_0, u_1, in_proj_w, conv_x_w, conv_z_w, x_proj_w, dt_proj_w,
           dt_proj_b, A_log, D, out_proj_w):
    b0 = u_0.shape[0]
    u_all = jnp.concatenate([u_0, u_1], axis=0)
    nb = u_all.shape[0]

    w_in_T = in_proj_w.T
    w_dl_T = jnp.dot(x_proj_w[:_DTR].T, dt_proj_w.T, precision=_HP)
    w_bc = x_proj_w[_DTR:]
    w_out_T = out_proj_w.T
    A_T = -jnp.exp(A_log).T
    aux = jnp.stack([conv_x_w[:, 0, 0], conv_x_w[:, 0, 1], conv_x_w[:, 0, 2],
                     conv_z_w[:, 0, 0], conv_z_w[:, 0, 1], conv_z_w[:, 0, 2],
                     dt_proj_b, D], axis=0)

    out_all = pl.pallas_call(
        _mamba_body,
        grid=(nb,),
        in_specs=[
            pl.BlockSpec((1, _L, _DM), lambda i: (i, 0, 0)),
            pl.BlockSpec((_DM, _DM), lambda i: (0, 0)),
            pl.BlockSpec((_DH, _DH), lambda i: (0, 0)),
            pl.BlockSpec((2 * _DS, _DH), lambda i: (0, 0)),
            pl.BlockSpec((_DM, _DM), lambda i: (0, 0)),
            pl.BlockSpec((_DS, _DH), lambda i: (0, 0)),
            pl.BlockSpec((8, _DH), lambda i: (0, 0)),
        ],
        out_specs=pl.BlockSpec((1, _L, _DM), lambda i: (i, 0, 0)),
        out_shape=jax.ShapeDtypeStruct((nb, _L, _DM), jnp.float32),
        scratch_shapes=[
            pltpu.VMEM((_L, _DM), jnp.float32),
            pltpu.VMEM((_L, _DH), jnp.float32),
            pltpu.VMEM((_L, _DH), jnp.float32),
            pltpu.VMEM((_L, _DH), jnp.float32),
            pltpu.VMEM((_L, _DH), jnp.float32),
            pltpu.VMEM((2 * _DS, _L), jnp.float32),
            pltpu.VMEM((_L, _DH), jnp.float32),
            pltpu.VMEM((2, _SCHUNK * _DS, _DH), jnp.float32),
        ],
        compiler_params=pltpu.CompilerParams(
            dimension_semantics=("parallel",),
            vmem_limit_bytes=56 * 1024 * 1024,
        ),
        name="mamba_vision_fused",
    )(u_all, w_in_T, w_dl_T, w_bc, w_out_T, A_T, aux)

    return out_all[:b0], out_all[b0:]


# speedup vs baseline: 1.1163x; 1.0038x over previous
import jax
import jax.numpy as jnp
from jax.experimental import pallas as pl
from jax.experimental.pallas import tpu as pltpu

_L = 4096
_DM = 256
_DH = 128
_DS = 16
_DTR = 16
_CHUNK = 512
_NC = _L // _CHUNK
_SCHUNK = 128
_NSC = _L // _SCHUNK

_HP = jax.lax.Precision.HIGHEST
_DP = jax.lax.Precision.DEFAULT


def _silu(v):
    return v * jax.lax.logistic(v)


def _mamba_body(u_ref, w_in_ref, w_dl_ref, w_bc_ref, w_out_ref, A_ref, aux_ref,
                o_ref, xz_s, xf_s, zf_s, dl_s, dxu_s, bct_s, y_s, a_s):
    for k in range(_NC):
        sl = slice(k * _CHUNK, (k + 1) * _CHUNK)
        xz_s[sl, :] = jnp.dot(u_ref[0, sl, :].astype(jnp.bfloat16),
                              w_in_ref[...].astype(jnp.bfloat16),
                              preferred_element_type=jnp.float32)

    w0x = aux_ref[0:1, :]
    w1x = aux_ref[1:2, :]
    w2x = aux_ref[2:3, :]
    w0z = aux_ref[3:4, :]
    w1z = aux_ref[4:5, :]
    w2z = aux_ref[5:6, :]
    zrow = jnp.zeros((1, _DH), jnp.float32)
    for k in range(_NC):
        sl = slice(k * _CHUNK, (k + 1) * _CHUNK)
        xc = xz_s[sl, 0:_DH]
        zc = xz_s[sl, _DH:_DM]
        if k == 0:
            px, pz = zrow, zrow
        else:
            prev = xz_s[k * _CHUNK - 8:k * _CHUNK, :]
            px, pz = prev[7:8, 0:_DH], prev[7:8, _DH:_DM]
        if k == _NC - 1:
            nx, nz = zrow, zrow
        else:
            nxt = xz_s[(k + 1) * _CHUNK:(k + 1) * _CHUNK + 8, :]
            nx, nz = nxt[0:1, 0:_DH], nxt[0:1, _DH:_DM]
        xl = jnp.concatenate([px, xc[:-1]], axis=0)
        xr = jnp.concatenate([xc[1:], nx], axis=0)
        zl = jnp.concatenate([pz, zc[:-1]], axis=0)
        zr = jnp.concatenate([zc[1:], nz], axis=0)
        xf_s[sl, :] = _silu(w0x * xl + w1x * xc + w2x * xr)
        zf_s[sl, :] = _silu(w0z * zl + w1z * zc + w2z * zr)

    dtb = aux_ref[6:7, :]
    for k in range(_NC):
        sl = slice(k * _CHUNK, (k + 1) * _CHUNK)
        xfc = xf_s[sl, :]
        dp = jnp.dot(xfc, w_dl_ref[...], precision=_HP) + dtb
        dl = jnp.where(dp > 20.0, dp,
                       jnp.log(1.0 + jnp.exp(jnp.minimum(dp, 20.0))))
        dl_s[sl, :] = dl
        dxu_s[sl, :] = dl * xfc
        bct_s[:, sl] = jax.lax.dot_general(
            w_bc_ref[...].astype(jnp.bfloat16), xfc.astype(jnp.bfloat16),
            (((1,), (1,)), ((), ())), preferred_element_type=jnp.float32)

    A_v = A_ref[...]

    def _fill_a(slot, cc):
        dblk = dl_s[pl.ds(pl.multiple_of(cc * _SCHUNK, _SCHUNK), _SCHUNK), :]
        for j in range(_SCHUNK):
            a_s[slot, j * _DS:(j + 1) * _DS, :] = jnp.exp(A_v * dblk[j:j + 1, :])

    _fill_a(0, 0)

    def chunk(c, h):
        base = pl.multiple_of(c * _SCHUNK, _SCHUNK)
        par = jax.lax.rem(c, 2)
        xblk = dxu_s[pl.ds(base, _SCHUNK), :]
        Bblk = bct_s[0:_DS, pl.ds(base, _SCHUNK)]
        Cblk = bct_s[_DS:2 * _DS, pl.ds(base, _SCHUNK)]
        _fill_a(1 - par, jnp.minimum(c + 1, _NSC - 1))
        for g in range(_SCHUNK // 8):
            rows = []
            for j in range(g * 8, (g + 1) * 8):
                bj = xblk[j:j + 1, :] * Bblk[:, j:j + 1]
                h = a_s[par, j * _DS:(j + 1) * _DS, :] * h + bj
                rows.append(jnp.sum(h * Cblk[:, j:j + 1], axis=0, keepdims=True))
            y_s[pl.ds(base + g * 8, 8), :] = jnp.concatenate(rows, axis=0)
        return h

    jax.lax.fori_loop(0, _NSC, chunk, jnp.zeros((_DS, _DH), jnp.float32))

    Dv = aux_ref[7:8, :]
    for k in range(_NC):
        sl = slice(k * _CHUNK, (k + 1) * _CHUNK)
        cat = jnp.concatenate([y_s[sl, :] + xf_s[sl, :] * Dv, zf_s[sl, :]], axis=1)
        o_ref[0, sl, :] = jnp.dot(cat.astype(jnp.bfloat16),
                                  w_out_ref[...].astype(jnp.bfloat16),
                                  preferred_element_type=jnp.float32)


def kernel(u_0, u_1, in_proj_w, conv_x_w, conv_z_w, x_proj_w, dt_proj_w,
           dt_proj_b, A_log, D, out_proj_w):
    b0 = u_0.shape[0]
    u_all = jnp.concatenate([u_0, u_1], axis=0)
    nb = u_all.shape[0]

    w_in_T = in_proj_w.T
    w_dl_T = jnp.dot(x_proj_w[:_DTR].T, dt_proj_w.T, precision=_HP)
    w_bc = x_proj_w[_DTR:]
    w_out_T = out_proj_w.T
    A_T = -jnp.exp(A_log).T
    aux = jnp.stack([conv_x_w[:, 0, 0], conv_x_w[:, 0, 1], conv_x_w[:, 0, 2],
                     conv_z_w[:, 0, 0], conv_z_w[:, 0, 1], conv_z_w[:, 0, 2],
                     dt_proj_b, D], axis=0)

    out_all = pl.pallas_call(
        _mamba_body,
        grid=(nb,),
        in_specs=[
            pl.BlockSpec((1, _L, _DM), lambda i: (i, 0, 0)),
            pl.BlockSpec((_DM, _DM), lambda i: (0, 0)),
            pl.BlockSpec((_DH, _DH), lambda i: (0, 0)),
            pl.BlockSpec((2 * _DS, _DH), lambda i: (0, 0)),
            pl.BlockSpec((_DM, _DM), lambda i: (0, 0)),
            pl.BlockSpec((_DS, _DH), lambda i: (0, 0)),
            pl.BlockSpec((8, _DH), lambda i: (0, 0)),
        ],
        out_specs=pl.BlockSpec((1, _L, _DM), lambda i: (i, 0, 0)),
        out_shape=jax.ShapeDtypeStruct((nb, _L, _DM), jnp.float32),
        scratch_shapes=[
            pltpu.VMEM((_L, _DM), jnp.float32),
            pltpu.VMEM((_L, _DH), jnp.float32),
            pltpu.VMEM((_L, _DH), jnp.float32),
            pltpu.VMEM((_L, _DH), jnp.float32),
            pltpu.VMEM((_L, _DH), jnp.float32),
            pltpu.VMEM((2 * _DS, _L), jnp.float32),
            pltpu.VMEM((_L, _DH), jnp.float32),
            pltpu.VMEM((2, _SCHUNK * _DS, _DH), jnp.float32),
        ],
        compiler_params=pltpu.CompilerParams(
            dimension_semantics=("parallel",),
            vmem_limit_bytes=56 * 1024 * 1024,
        ),
        name="mamba_vision_fused",
    )(u_all, w_in_T, w_dl_T, w_bc, w_out_T, A_T, aux)

    return out_all[:b0], out_all[b0:]


# speedup vs baseline: 1.2719x; 1.1393x over previous
import jax
import jax.numpy as jnp
from jax.experimental import pallas as pl
from jax.experimental.pallas import tpu as pltpu

_L = 4096
_DM = 256
_DH = 128
_DS = 16
_DTR = 16
_CHUNK = 512
_NC = _L // _CHUNK
_SCHUNK = 128
_NSC = _L // _SCHUNK

_HP = jax.lax.Precision.HIGHEST
_BF = jnp.bfloat16


def _silu(v):
    return v * jax.lax.logistic(v)


def _mamba_body(u_ref, w_in_ref, w_dl_ref, w_bc_ref, w_out_ref, A_ref, aux_ref,
                o_ref, xzy_s, xf_s, zf_s, dl_s, dxu_s, bct_s, a_s):
    w0x = aux_ref[0:1, :]
    w1x = aux_ref[1:2, :]
    w2x = aux_ref[2:3, :]
    w0z = aux_ref[3:4, :]
    w1z = aux_ref[4:5, :]
    w2z = aux_ref[5:6, :]
    dtb = aux_ref[6:7, :]
    zrow = jnp.zeros((1, _DH), jnp.float32)

    for un in range(2):
        for k in range(_NC):
            sl = slice(k * _CHUNK, (k + 1) * _CHUNK)
            xzy_s[sl, :] = jnp.dot(u_ref[un, sl, :].astype(_BF),
                                   w_in_ref[...].astype(_BF),
                                   preferred_element_type=jnp.float32)

        for k in range(_NC):
            sl = slice(k * _CHUNK, (k + 1) * _CHUNK)
            xc = xzy_s[sl, 0:_DH]
            zc = xzy_s[sl, _DH:_DM]
            if k == 0:
                px, pz = zrow, zrow
            else:
                prev = xzy_s[k * _CHUNK - 8:k * _CHUNK, :]
                px, pz = prev[7:8, 0:_DH], prev[7:8, _DH:_DM]
            if k == _NC - 1:
                nx, nz = zrow, zrow
            else:
                nxt = xzy_s[(k + 1) * _CHUNK:(k + 1) * _CHUNK + 8, :]
                nx, nz = nxt[0:1, 0:_DH], nxt[0:1, _DH:_DM]
            xl = jnp.concatenate([px, xc[:-1]], axis=0)
            xr = jnp.concatenate([xc[1:], nx], axis=0)
            zl = jnp.concatenate([pz, zc[:-1]], axis=0)
            zr = jnp.concatenate([zc[1:], nz], axis=0)
            xf_s[un, sl, :] = _silu(w0x * xl + w1x * xc + w2x * xr)
            zf_s[un, sl, :] = _silu(w0z * zl + w1z * zc + w2z * zr)

        for k in range(_NC):
            sl = slice(k * _CHUNK, (k + 1) * _CHUNK)
            xfc = xf_s[un, sl, :]
            dp = jnp.dot(xfc, w_dl_ref[...], precision=_HP) + dtb
            dl = jnp.where(dp > 20.0, dp,
                           jnp.log(1.0 + jnp.exp(jnp.minimum(dp, 20.0))))
            dl_s[un, sl, :] = dl
            dxu_s[un, sl, :] = dl * xfc
            bct_s[un, :, sl] = jax.lax.dot_general(
                w_bc_ref[...].astype(_BF), xfc.astype(_BF),
                (((1,), (1,)), ((), ())), preferred_element_type=jnp.float32)

    A_v = A_ref[...]

    def chunk(c, carry):
        h0, h1 = carry
        base = pl.multiple_of(c * _SCHUNK, _SCHUNK)
        dblk0 = dl_s[0, pl.ds(base, _SCHUNK), :]
        dblk1 = dl_s[1, pl.ds(base, _SCHUNK), :]
        for j in range(_SCHUNK):
            a_s[0, j * _DS:(j + 1) * _DS, :] = jnp.exp(A_v * dblk0[j:j + 1, :])
            a_s[1, j * _DS:(j + 1) * _DS, :] = jnp.exp(A_v * dblk1[j:j + 1, :])
        xblk0 = dxu_s[0, pl.ds(base, _SCHUNK), :]
        xblk1 = dxu_s[1, pl.ds(base, _SCHUNK), :]
        Bblk0 = bct_s[0, 0:_DS, pl.ds(base, _SCHUNK)]
        Bblk1 = bct_s[1, 0:_DS, pl.ds(base, _SCHUNK)]
        Cblk0 = bct_s[0, _DS:2 * _DS, pl.ds(base, _SCHUNK)]
        Cblk1 = bct_s[1, _DS:2 * _DS, pl.ds(base, _SCHUNK)]
        for g in range(_SCHUNK // 8):
            rows0 = []
            rows1 = []
            for j in range(g * 8, (g + 1) * 8):
                b0 = xblk0[j:j + 1, :] * Bblk0[:, j:j + 1]
                b1 = xblk1[j:j + 1, :] * Bblk1[:, j:j + 1]
                h0 = a_s[0, j * _DS:(j + 1) * _DS, :] * h0 + b0
                h1 = a_s[1, j * _DS:(j + 1) * _DS, :] * h1 + b1
                rows0.append(jnp.sum(h0 * Cblk0[:, j:j + 1], axis=0,
                                     keepdims=True))
                rows1.append(jnp.sum(h1 * Cblk1[:, j:j + 1], axis=0,
                                     keepdims=True))
            xzy_s[pl.ds(base + g * 8, 8), 0:_DH] = jnp.concatenate(rows0,
                                                                   axis=0)
            xzy_s[pl.ds(base + g * 8, 8), _DH:_DM] = jnp.concatenate(rows1,
                                                                     axis=0)
        return (h0, h1)

    hz = jnp.zeros((_DS, _DH), jnp.float32)
    jax.lax.fori_loop(0, _NSC, chunk, (hz, hz))

    Dv = aux_ref[7:8, :]
    for un in range(2):
        for k in range(_NC):
            sl = slice(k * _CHUNK, (k + 1) * _CHUNK)
            yk = xzy_s[sl, un * _DH:(un + 1) * _DH]
            cat = jnp.concatenate([yk + xf_s[un, sl, :] * Dv,
                                   zf_s[un, sl, :]], axis=1)
            o_ref[un, sl, :] = jnp.dot(cat.astype(_BF),
                                       w_out_ref[...].astype(_BF),
                                       preferred_element_type=jnp.float32)


def kernel(u_0, u_1, in_proj_w, conv_x_w, conv_z_w, x_proj_w, dt_proj_w,
           dt_proj_b, A_log, D, out_proj_w):
    b0 = u_0.shape[0]
    u_all = jnp.concatenate([u_0, u_1], axis=0)
    nb = u_all.shape[0]

    w_in_T = in_proj_w.T
    w_dl_T = jnp.dot(x_proj_w[:_DTR].T, dt_proj_w.T, precision=_HP)
    w_bc = x_proj_w[_DTR:]
    w_out_T = out_proj_w.T
    A_T = -jnp.exp(A_log).T
    aux = jnp.stack([conv_x_w[:, 0, 0], conv_x_w[:, 0, 1], conv_x_w[:, 0, 2],
                     conv_z_w[:, 0, 0], conv_z_w[:, 0, 1], conv_z_w[:, 0, 2],
                     dt_proj_b, D], axis=0)

    out_all = pl.pallas_call(
        _mamba_body,
        grid=(nb // 2,),
        in_specs=[
            pl.BlockSpec((2, _L, _DM), lambda i: (i, 0, 0)),
            pl.BlockSpec((_DM, _DM), lambda i: (0, 0)),
            pl.BlockSpec((_DH, _DH), lambda i: (0, 0)),
            pl.BlockSpec((2 * _DS, _DH), lambda i: (0, 0)),
            pl.BlockSpec((_DM, _DM), lambda i: (0, 0)),
            pl.BlockSpec((_DS, _DH), lambda i: (0, 0)),
            pl.BlockSpec((8, _DH), lambda i: (0, 0)),
        ],
        out_specs=pl.BlockSpec((2, _L, _DM), lambda i: (i, 0, 0)),
        out_shape=jax.ShapeDtypeStruct((nb, _L, _DM), jnp.float32),
        scratch_shapes=[
            pltpu.VMEM((_L, _DM), jnp.float32),
            pltpu.VMEM((2, _L, _DH), jnp.float32),
            pltpu.VMEM((2, _L, _DH), jnp.float32),
            pltpu.VMEM((2, _L, _DH), jnp.float32),
            pltpu.VMEM((2, _L, _DH), jnp.float32),
            pltpu.VMEM((2, 2 * _DS, _L), jnp.float32),
            pltpu.VMEM((2, _SCHUNK * _DS, _DH), jnp.float32),
        ],
        compiler_params=pltpu.CompilerParams(
            dimension_semantics=("parallel",),
            vmem_limit_bytes=58 * 1024 * 1024,
        ),
        name="mamba_vision_fused",
    )(u_all, w_in_T, w_dl_T, w_bc, w_out_T, A_T, aux)

    return out_all[:b0], out_all[b0:]


# speedup vs baseline: 1.3104x; 1.0303x over previous
import jax
import jax.numpy as jnp
from jax.experimental import pallas as pl
from jax.experimental.pallas import tpu as pltpu

_L = 4096
_DM = 256
_DH = 128
_DS = 16
_DTR = 16
_CHUNK = 512
_NC = _L // _CHUNK
_SCHUNK = 128
_NSC = _L // _SCHUNK

_HP = jax.lax.Precision.HIGHEST
_BF = jnp.bfloat16


def _silu(v):
    return v * jax.lax.logistic(v)


def _mamba_body(u_ref, w_in_ref, w_dl_ref, w_bc_ref, w_out_ref, A_ref, aux_ref,
                o_ref, xzy_s, xf_s, zf_s, dl_s, dxu_s, bct_s, a_s):
    w0x = aux_ref[0:1, :]
    w1x = aux_ref[1:2, :]
    w2x = aux_ref[2:3, :]
    w0z = aux_ref[3:4, :]
    w1z = aux_ref[4:5, :]
    w2z = aux_ref[5:6, :]
    dtb = aux_ref[6:7, :]
    zrow = jnp.zeros((1, _DH), jnp.float32)

    for un in range(2):
        for k in range(_NC):
            sl = slice(k * _CHUNK, (k + 1) * _CHUNK)
            xzy_s[sl, :] = jnp.dot(u_ref[un, sl, :].astype(_BF),
                                   w_in_ref[...].astype(_BF),
                                   preferred_element_type=jnp.float32)

        for k in range(_NC):
            sl = slice(k * _CHUNK, (k + 1) * _CHUNK)
            xc = xzy_s[sl, 0:_DH]
            zc = xzy_s[sl, _DH:_DM]
            if k == 0:
                px, pz = zrow, zrow
            else:
                prev = xzy_s[k * _CHUNK - 8:k * _CHUNK, :]
                px, pz = prev[7:8, 0:_DH], prev[7:8, _DH:_DM]
            if k == _NC - 1:
                nx, nz = zrow, zrow
            else:
                nxt = xzy_s[(k + 1) * _CHUNK:(k + 1) * _CHUNK + 8, :]
                nx, nz = nxt[0:1, 0:_DH], nxt[0:1, _DH:_DM]
            xl = jnp.concatenate([px, xc[:-1]], axis=0)
            xr = jnp.concatenate([xc[1:], nx], axis=0)
            zl = jnp.concatenate([pz, zc[:-1]], axis=0)
            zr = jnp.concatenate([zc[1:], nz], axis=0)
            xf_s[un, sl, :] = _silu(w0x * xl + w1x * xc + w2x * xr)
            zf_s[un, sl, :] = _silu(w0z * zl + w1z * zc + w2z * zr)

        for k in range(_NC):
            sl = slice(k * _CHUNK, (k + 1) * _CHUNK)
            xfc = xf_s[un, sl, :]
            dp = jnp.dot(xfc.astype(_BF), w_dl_ref[...].astype(_BF),
                         preferred_element_type=jnp.float32) + dtb
            dl = jnp.where(dp > 20.0, dp,
                           jnp.log(1.0 + jnp.exp(jnp.minimum(dp, 20.0))))
            dl_s[un, sl, :] = dl
            dxu_s[un, sl, :] = dl * xfc
            bct_s[un, :, sl] = jax.lax.dot_general(
                w_bc_ref[...].astype(_BF), xfc.astype(_BF),
                (((1,), (1,)), ((), ())), preferred_element_type=jnp.float32)

    A_v = A_ref[...]

    def chunk(c, carry):
        h0, h1 = carry
        base = pl.multiple_of(c * _SCHUNK, _SCHUNK)
        dblk0 = dl_s[0, pl.ds(base, _SCHUNK), :]
        dblk1 = dl_s[1, pl.ds(base, _SCHUNK), :]
        for j in range(_SCHUNK):
            a_s[0, j * _DS:(j + 1) * _DS, :] = jnp.exp(A_v * dblk0[j:j + 1, :])
            a_s[1, j * _DS:(j + 1) * _DS, :] = jnp.exp(A_v * dblk1[j:j + 1, :])
        xblk0 = dxu_s[0, pl.ds(base, _SCHUNK), :]
        xblk1 = dxu_s[1, pl.ds(base, _SCHUNK), :]
        Bblk0 = bct_s[0, 0:_DS, pl.ds(base, _SCHUNK)]
        Bblk1 = bct_s[1, 0:_DS, pl.ds(base, _SCHUNK)]
        Cblk0 = bct_s[0, _DS:2 * _DS, pl.ds(base, _SCHUNK)]
        Cblk1 = bct_s[1, _DS:2 * _DS, pl.ds(base, _SCHUNK)]
        for g in range(_SCHUNK // 8):
            rows0 = []
            rows1 = []
            for j in range(g * 8, (g + 1) * 8):
                b0 = xblk0[j:j + 1, :] * Bblk0[:, j:j + 1]
                b1 = xblk1[j:j + 1, :] * Bblk1[:, j:j + 1]
                h0 = a_s[0, j * _DS:(j + 1) * _DS, :] * h0 + b0
                h1 = a_s[1, j * _DS:(j + 1) * _DS, :] * h1 + b1
                rows0.append(jnp.sum(h0 * Cblk0[:, j:j + 1], axis=0,
                                     keepdims=True))
                rows1.append(jnp.sum(h1 * Cblk1[:, j:j + 1], axis=0,
                                     keepdims=True))
            xzy_s[pl.ds(base + g * 8, 8), 0:_DH] = jnp.concatenate(rows0,
                                                                   axis=0)
            xzy_s[pl.ds(base + g * 8, 8), _DH:_DM] = jnp.concatenate(rows1,
                                                                     axis=0)
        return (h0, h1)

    hz = jnp.zeros((_DS, _DH), jnp.float32)
    jax.lax.fori_loop(0, _NSC, chunk, (hz, hz))

    Dv = aux_ref[7:8, :]
    for un in range(2):
        for k in range(_NC):
            sl = slice(k * _CHUNK, (k + 1) * _CHUNK)
            yk = xzy_s[sl, un * _DH:(un + 1) * _DH]
            cat = jnp.concatenate([yk + xf_s[un, sl, :] * Dv,
                                   zf_s[un, sl, :]], axis=1)
            o_ref[un, sl, :] = jnp.dot(cat.astype(_BF),
                                       w_out_ref[...].astype(_BF),
                                       preferred_element_type=jnp.float32)


def kernel(u_0, u_1, in_proj_w, conv_x_w, conv_z_w, x_proj_w, dt_proj_w,
           dt_proj_b, A_log, D, out_proj_w):
    b0 = u_0.shape[0]
    u_all = jnp.concatenate([u_0, u_1], axis=0)
    nb = u_all.shape[0]

    w_in_T = in_proj_w.T
    w_dl_T = jnp.dot(x_proj_w[:_DTR].T, dt_proj_w.T, precision=_HP)
    w_bc = x_proj_w[_DTR:]
    w_out_T = out_proj_w.T
    A_T = -jnp.exp(A_log).T
    aux = jnp.stack([conv_x_w[:, 0, 0], conv_x_w[:, 0, 1], conv_x_w[:, 0, 2],
                     conv_z_w[:, 0, 0], conv_z_w[:, 0, 1], conv_z_w[:, 0, 2],
                     dt_proj_b, D], axis=0)

    out_all = pl.pallas_call(
        _mamba_body,
        grid=(nb // 2,),
        in_specs=[
            pl.BlockSpec((2, _L, _DM), lambda i: (i, 0, 0)),
            pl.BlockSpec((_DM, _DM), lambda i: (0, 0)),
            pl.BlockSpec((_DH, _DH), lambda i: (0, 0)),
            pl.BlockSpec((2 * _DS, _DH), lambda i: (0, 0)),
            pl.BlockSpec((_DM, _DM), lambda i: (0, 0)),
            pl.BlockSpec((_DS, _DH), lambda i: (0, 0)),
            pl.BlockSpec((8, _DH), lambda i: (0, 0)),
        ],
        out_specs=pl.BlockSpec((2, _L, _DM), lambda i: (i, 0, 0)),
        out_shape=jax.ShapeDtypeStruct((nb, _L, _DM), jnp.float32),
        scratch_shapes=[
            pltpu.VMEM((_L, _DM), jnp.float32),
            pltpu.VMEM((2, _L, _DH), jnp.float32),
            pltpu.VMEM((2, _L, _DH), jnp.float32),
            pltpu.VMEM((2, _L, _DH), jnp.float32),
            pltpu.VMEM((2, _L, _DH), jnp.float32),
            pltpu.VMEM((2, 2 * _DS, _L), jnp.float32),
            pltpu.VMEM((2, _SCHUNK * _DS, _DH), jnp.float32),
        ],
        compiler_params=pltpu.CompilerParams(
            dimension_semantics=("parallel",),
            vmem_limit_bytes=58 * 1024 * 1024,
        ),
        name="mamba_vision_fused",
    )(u_all, w_in_T, w_dl_T, w_bc, w_out_T, A_T, aux)

    return out_all[:b0], out_all[b0:]
